# Initial kernel scaffold; baseline (speedup 1.0000x reference)
import jax
import jax.numpy as jnp
from jax.experimental import pallas as pl


def kernel(points, sa_params, fp_params):
    raise NotImplementedError("write your pallas kernel here")


# speedup vs baseline: 5.7442x; 5.7442x over previous
import functools

import jax
import jax.numpy as jnp
from jax import lax
from jax.experimental import pallas as pl
from jax.experimental.pallas import tpu as pltpu

_B = 4
_NPOINTS = [4096, 2048, 1024, 512]
_RADII = [0.05, 0.1, 0.2, 0.3]
_NSAMPLE = 32

_HI = jax.lax.Precision.HIGHEST
_F32 = jnp.float32


def _dot(a, b):
    return lax.dot_general(a, b, (((1,), (0,)), ((), ())),
                           precision=_HI, preferred_element_type=_F32)


def _fps_body(npoint, x_ref, y_ref, z_ref, cx_ref, cy_ref, cz_ref):
    b, n = x_ref.shape
    X = x_ref[...]
    Y = y_ref[...]
    Z = z_ref[...]
    iota = lax.broadcasted_iota(jnp.int32, (b, n), 1)
    l128 = lax.broadcasted_iota(jnp.int32, (b, 128), 1)

    def body(i, carry):
        far, dists, bx, by, bz = carry
        onehot = iota == far
        cx = jnp.sum(jnp.where(onehot, X, 0.0), axis=1, keepdims=True)
        cy = jnp.sum(jnp.where(onehot, Y, 0.0), axis=1, keepdims=True)
        cz = jnp.sum(jnp.where(onehot, Z, 0.0), axis=1, keepdims=True)
        bx = jnp.where(l128 == 127, cx, jnp.roll(bx, -1, axis=1))
        by = jnp.where(l128 == 127, cy, jnp.roll(by, -1, axis=1))
        bz = jnp.where(l128 == 127, cz, jnp.roll(bz, -1, axis=1))

        @pl.when(i % 128 == 127)
        def _flush():
            base = pl.multiple_of(i - 127, 128)
            cx_ref[:, pl.ds(base, 128)] = bx
            cy_ref[:, pl.ds(base, 128)] = by
            cz_ref[:, pl.ds(base, 128)] = bz

        dx = X - cx
        dy = Y - cy
        dz = Z - cz
        d = (dx * dx + dy * dy) + dz * dz
        dists = jnp.minimum(dists, d)
        m = jnp.max(dists, axis=1, keepdims=True)
        far = jnp.min(jnp.where(dists == m, iota, n), axis=1, keepdims=True)
        return far, dists, bx, by, bz

    far0 = jnp.zeros((b, 1), jnp.int32)
    d0 = jnp.full((b, n), 1e10, _F32)
    buf0 = jnp.zeros((b, 128), _F32)
    lax.fori_loop(0, npoint, body, (far0, d0, buf0, buf0, buf0))


def _fps(xyz, npoint):
    b, n, _ = xyz.shape
    X = xyz[..., 0]
    Y = xyz[..., 1]
    Z = xyz[..., 2]
    out_sd = jax.ShapeDtypeStruct((b, npoint), _F32)
    cx, cy, cz = pl.pallas_call(
        functools.partial(_fps_body, npoint),
        out_shape=(out_sd, out_sd, out_sd),
    )(X, Y, Z)
    return jnp.stack([cx, cy, cz], axis=-1)


def _linear_body(x_ref, w_ref, b_ref, o_ref):
    x = x_ref[...].reshape(x_ref.shape[1], x_ref.shape[2])
    o = _dot(x, w_ref[...]) + b_ref[...]
    o_ref[...] = o.reshape(o_ref.shape)


def _linear(x, w, bias):
    b, n, cin = x.shape
    d = w.shape[1]
    return pl.pallas_call(
        _linear_body,
        grid=(b,),
        in_specs=[
            pl.BlockSpec((1, n, cin), lambda i: (i, 0, 0)),
            pl.BlockSpec((cin, d), lambda i: (0, 0)),
            pl.BlockSpec((1, d), lambda i: (0, 0)),
        ],
        out_specs=pl.BlockSpec((1, n, d), lambda i: (i, 0, 0)),
        out_shape=jax.ShapeDtypeStruct((b, n, d), _F32),
    )(x, w, bias.reshape(1, d))


_CH = 128


def _sa_body(r2, nsample, x_ref, y_ref, z_ref, qx_ref, qy_ref, qz_ref,
             fx_ref, qw_ref, w2_ref, b2_ref, w3_ref, b3_ref, o_ref,
             m_s, r_s):
    qb = qx_ref.shape[2]
    n = x_ref.shape[2]
    d1 = fx_ref.shape[2]
    d3 = o_ref.shape[2]

    x = x_ref[...].reshape(1, n)
    y = y_ref[...].reshape(1, n)
    z = z_ref[...].reshape(1, n)
    qx = qx_ref[...].reshape(qb, 1)
    qy = qy_ref[...].reshape(qb, 1)
    qz = qz_ref[...].reshape(qb, 1)

    dx = qx - x
    dy = qy - y
    dz = qz - z
    d2 = (dx * dx + dy * dy) + dz * dz
    m_s[...] = (d2 <= r2).astype(_F32)

    u = (lax.broadcasted_iota(jnp.int32, (_CH, _CH), 0)
         <= lax.broadcasted_iota(jnp.int32, (_CH, _CH), 1)).astype(_F32)

    def ch_body(c, base):
        mc = m_s[:, pl.ds(c * _CH, _CH)]
        loc = lax.dot_general(mc, u, (((1,), (0,)), ((), ())),
                              preferred_element_type=_F32)
        r_s[:, pl.ds(c * _CH, _CH)] = loc + base
        return base + loc[:, _CH - 1:_CH]

    lax.fori_loop(0, n // _CH, ch_body, jnp.zeros((qb, 1), _F32))

    count = r_s[:, n - 1:n]
    r_s[...] = jnp.where(m_s[...] > 0.0, r_s[...], 0.0)

    fx = fx_ref[...].reshape(n, d1)
    qw = qw_ref[...].reshape(qb, d1)
    w2 = w2_ref[...]
    b2 = b2_ref[...]
    w3 = w3_ref[...]
    b3 = b3_ref[...]

    def k_body(k, acc):
        kf = (k + 1).astype(_F32)
        oh = (r_s[...] == kf).astype(_F32)
        g = _dot(oh, fx)
        h1 = jnp.maximum(g - qw, 0.0)
        h2 = jnp.maximum(_dot(h1, w2) + b2, 0.0)
        h3 = jnp.maximum(_dot(h2, w3) + b3, 0.0)
        valid = count > k.astype(_F32)
        return jnp.maximum(acc, jnp.where(valid, h3, -jnp.inf))

    acc0 = jnp.full((qb, d3), -jnp.inf, _F32)
    acc = lax.fori_loop(0, nsample, k_body, acc0)
    o_ref[...] = acc.reshape(1, qb, d3)


def _sa(xyz, new_xyz, fx, qw, w2, b2, w3, b3, radius, qblk):
    b, n, _ = xyz.shape
    q = new_xyz.shape[1]
    d1 = fx.shape[2]
    d2 = w2.shape[1]
    d3 = w3.shape[1]
    X = xyz[..., 0].reshape(b, 1, n)
    Y = xyz[..., 1].reshape(b, 1, n)
    Z = xyz[..., 2].reshape(b, 1, n)
    nblk = q // qblk
    QX = new_xyz[..., 0].reshape(b, nblk, qblk, 1)
    QY = new_xyz[..., 1].reshape(b, nblk, qblk, 1)
    QZ = new_xyz[..., 2].reshape(b, nblk, qblk, 1)
    grid = (b, nblk)
    return pl.pallas_call(
        functools.partial(_sa_body, radius * radius, _NSAMPLE),
        grid=grid,
        in_specs=[
            pl.BlockSpec((1, 1, n), lambda i, j: (i, 0, 0)),
            pl.BlockSpec((1, 1, n), lambda i, j: (i, 0, 0)),
            pl.BlockSpec((1, 1, n), lambda i, j: (i, 0, 0)),
            pl.BlockSpec((1, 1, qblk, 1), lambda i, j: (i, j, 0, 0)),
            pl.BlockSpec((1, 1, qblk, 1), lambda i, j: (i, j, 0, 0)),
            pl.BlockSpec((1, 1, qblk, 1), lambda i, j: (i, j, 0, 0)),
            pl.BlockSpec((1, n, d1), lambda i, j: (i, 0, 0)),
            pl.BlockSpec((1, qblk, d1), lambda i, j: (i, j, 0)),
            pl.BlockSpec((d1, d2), lambda i, j: (0, 0)),
            pl.BlockSpec((1, d2), lambda i, j: (0, 0)),
            pl.BlockSpec((d2, d3), lambda i, j: (0, 0)),
            pl.BlockSpec((1, d3), lambda i, j: (0, 0)),
        ],
        out_specs=pl.BlockSpec((1, qblk, d3), lambda i, j: (i, j, 0)),
        out_shape=jax.ShapeDtypeStruct((b, q, d3), _F32),
        scratch_shapes=[pltpu.VMEM((qblk, n), _F32),
                        pltpu.VMEM((qblk, n), _F32)],
    )(X, Y, Z, QX, QY, QZ, fx, qw, w2, b2.reshape(1, d2),
      w3, b3.reshape(1, d3))


def _fp_body(ux_ref, uy_ref, uz_ref, kx_ref, ky_ref, kz_ref,
             f_ref, w_ref, b_ref, o_ref):
    qb = ux_ref.shape[2]
    n = kx_ref.shape[2]
    c = f_ref.shape[2]

    ux = ux_ref[...].reshape(qb, 1)
    uy = uy_ref[...].reshape(qb, 1)
    uz = uz_ref[...].reshape(qb, 1)
    kx = kx_ref[...].reshape(1, n)
    ky = ky_ref[...].reshape(1, n)
    kz = kz_ref[...].reshape(1, n)

    dx = ux - kx
    dy = uy - ky
    dz = uz - kz
    d2 = (dx * dx + dy * dy) + dz * dz
    iota = lax.broadcasted_iota(jnp.int32, (qb, n), 1)

    m1 = jnp.min(d2, axis=1, keepdims=True)
    i1 = jnp.min(jnp.where(d2 == m1, iota, n), axis=1, keepdims=True)
    d2b = jnp.where(iota == i1, 1e30, d2)
    m2 = jnp.min(d2b, axis=1, keepdims=True)
    i2 = jnp.min(jnp.where(d2b == m2, iota, n), axis=1, keepdims=True)
    d2c = jnp.where(iota == i2, 1e30, d2b)
    m3 = jnp.min(d2c, axis=1, keepdims=True)
    i3 = jnp.min(jnp.where(d2c == m3, iota, n), axis=1, keepdims=True)

    w1 = 1.0 / (m1 + 1e-8)
    w2 = 1.0 / (m2 + 1e-8)
    w3 = 1.0 / (m3 + 1e-8)
    tot = (w1 + w2) + w3
    w1 = w1 / tot
    w2 = w2 / tot
    w3 = w3 / tot

    s = (jnp.where(iota == i1, w1, 0.0) + jnp.where(iota == i2, w2, 0.0)
         + jnp.where(iota == i3, w3, 0.0))
    feats = f_ref[...].reshape(n, c)
    interp = _dot(s, feats)
    up = jnp.maximum(_dot(interp, w_ref[...]) + b_ref[...], 0.0)
    o_ref[...] = up.reshape(o_ref.shape)


def _fp(unknown, known, known_feats, w, bias, qblk):
    b, nq, _ = unknown.shape
    n = known.shape[1]
    c = known_feats.shape[2]
    d = w.shape[1]
    nblk = nq // qblk
    UX = unknown[..., 0].reshape(b, nblk, qblk, 1)
    UY = unknown[..., 1].reshape(b, nblk, qblk, 1)
    UZ = unknown[..., 2].reshape(b, nblk, qblk, 1)
    KX = known[..., 0].reshape(b, 1, n)
    KY = known[..., 1].reshape(b, 1, n)
    KZ = known[..., 2].reshape(b, 1, n)
    grid = (b, nblk)
    return pl.pallas_call(
        _fp_body,
        grid=grid,
        in_specs=[
            pl.BlockSpec((1, 1, qblk, 1), lambda i, j: (i, j, 0, 0)),
            pl.BlockSpec((1, 1, qblk, 1), lambda i, j: (i, j, 0, 0)),
            pl.BlockSpec((1, 1, qblk, 1), lambda i, j: (i, j, 0, 0)),
            pl.BlockSpec((1, 1, n), lambda i, j: (i, 0, 0)),
            pl.BlockSpec((1, 1, n), lambda i, j: (i, 0, 0)),
            pl.BlockSpec((1, 1, n), lambda i, j: (i, 0, 0)),
            pl.BlockSpec((1, n, c), lambda i, j: (i, 0, 0)),
            pl.BlockSpec((c, d), lambda i, j: (0, 0)),
            pl.BlockSpec((1, d), lambda i, j: (0, 0)),
        ],
        out_specs=pl.BlockSpec((1, qblk, d), lambda i, j: (i, j, 0)),
        out_shape=jax.ShapeDtypeStruct((b, nq, d), _F32),
    )(UX, UY, UZ, KX, KY, KZ, known_feats, w, bias.reshape(1, d))


def kernel(points, sa_params, fp_params):
    xyz0 = points[..., :3]
    l_xyz = [xyz0]
    l_feats = [None]
    for k in range(4):
        xyz = l_xyz[k]
        feats = l_feats[k]
        (w1, b1), (w2, b2), (w3, b3) = sa_params[k]
        new_xyz = _fps(xyz, _NPOINTS[k])
        if feats is None:
            cand = xyz
        else:
            cand = jnp.concatenate([xyz, feats], axis=-1)
        fx = _linear(cand, w1, b1)
        qw = _linear(new_xyz, w1[:3], jnp.zeros((w1.shape[1],), _F32))
        qblk = min(_NPOINTS[k], 256)
        nf = _sa(xyz, new_xyz, fx, qw, w2, b2, w3, b3, _RADII[k], qblk)
        l_xyz.append(new_xyz)
        l_feats.append(nf)
    ups = []
    for k in range(3):
        (w, b), = fp_params[k]
        up = _fp(xyz0, l_xyz[k + 2], l_feats[k + 2], w, b, 512)
        ups.append(up)
    return jnp.concatenate([xyz0, l_feats[1]] + ups, axis=-1)


# speedup vs baseline: 13.9182x; 2.4230x over previous
import functools

import jax
import jax.numpy as jnp
from jax import lax
from jax.experimental import pallas as pl
from jax.experimental.pallas import tpu as pltpu

_B = 4
_NPOINTS = [4096, 2048, 1024, 512]
_RADII = [0.05, 0.1, 0.2, 0.3]
_NSAMPLE = 32

_F32 = jnp.float32


def _dot(a, b, prec=jax.lax.Precision.DEFAULT):
    return lax.dot_general(a, b, (((1,), (0,)), ((), ())),
                           precision=prec, preferred_element_type=_F32)


def _split(x):
    hi = x.astype(jnp.bfloat16).astype(_F32)
    return hi, x - hi


def _mm3(a, bh, bl):
    ah, al = _split(a)
    return _dot(ah, bh) + (_dot(ah, bl) + _dot(al, bh))


def _fps_body(npoint, x_ref, y_ref, z_ref, cx_ref, cy_ref, cz_ref):
    b, n = x_ref.shape
    X = x_ref[...]
    Y = y_ref[...]
    Z = z_ref[...]
    iota = lax.broadcasted_iota(jnp.int32, (b, n), 1)
    l128 = lax.broadcasted_iota(jnp.int32, (b, 128), 1)

    def body(i, carry):
        far, dists, bx, by, bz = carry
        onehot = iota == far
        cx = jnp.sum(jnp.where(onehot, X, 0.0), axis=1, keepdims=True)
        cy = jnp.sum(jnp.where(onehot, Y, 0.0), axis=1, keepdims=True)
        cz = jnp.sum(jnp.where(onehot, Z, 0.0), axis=1, keepdims=True)
        bx = jnp.where(l128 == 127, cx, jnp.roll(bx, -1, axis=1))
        by = jnp.where(l128 == 127, cy, jnp.roll(by, -1, axis=1))
        bz = jnp.where(l128 == 127, cz, jnp.roll(bz, -1, axis=1))

        @pl.when(i % 128 == 127)
        def _flush():
            base = pl.multiple_of(i - 127, 128)
            cx_ref[:, pl.ds(base, 128)] = bx
            cy_ref[:, pl.ds(base, 128)] = by
            cz_ref[:, pl.ds(base, 128)] = bz

        dx = X - cx
        dy = Y - cy
        dz = Z - cz
        d = (dx * dx + dy * dy) + dz * dz
        dists = jnp.minimum(dists, d)
        m = jnp.max(dists, axis=1, keepdims=True)
        far = jnp.min(jnp.where(dists == m, iota, n), axis=1, keepdims=True)
        return far, dists, bx, by, bz

    far0 = jnp.zeros((b, 1), jnp.int32)
    d0 = jnp.full((b, n), 1e10, _F32)
    buf0 = jnp.zeros((b, 128), _F32)
    lax.fori_loop(0, npoint, body, (far0, d0, buf0, buf0, buf0))


def _fps(xyz, npoint):
    b, n, _ = xyz.shape
    X = xyz[..., 0]
    Y = xyz[..., 1]
    Z = xyz[..., 2]
    out_sd = jax.ShapeDtypeStruct((b, npoint), _F32)
    cx, cy, cz = pl.pallas_call(
        functools.partial(_fps_body, npoint),
        out_shape=(out_sd, out_sd, out_sd),
    )(X, Y, Z)
    return jnp.stack([cx, cy, cz], axis=-1)


def _linear_body(x_ref, w_ref, b_ref, o_ref):
    x = x_ref[...].reshape(x_ref.shape[1], x_ref.shape[2])
    wh, wl = _split(w_ref[...])
    o = _mm3(x, wh, wl) + b_ref[...]
    o_ref[...] = o.reshape(o_ref.shape)


def _linear(x, w, bias):
    b, n, cin = x.shape
    d = w.shape[1]
    return pl.pallas_call(
        _linear_body,
        grid=(b,),
        in_specs=[
            pl.BlockSpec((1, n, cin), lambda i: (i, 0, 0)),
            pl.BlockSpec((cin, d), lambda i: (0, 0)),
            pl.BlockSpec((1, d), lambda i: (0, 0)),
        ],
        out_specs=pl.BlockSpec((1, n, d), lambda i: (i, 0, 0)),
        out_shape=jax.ShapeDtypeStruct((b, n, d), _F32),
    )(x, w, bias.reshape(1, d))


_CH = 128


def _sa_body(r2, nsample, x_ref, y_ref, z_ref, qx_ref, qy_ref, qz_ref,
             fx_ref, qw_ref, w2_ref, b2_ref, w3_ref, b3_ref, o_ref,
             m_s, r_s):
    qb = qx_ref.shape[2]
    n = x_ref.shape[2]
    d1 = fx_ref.shape[2]
    d3 = o_ref.shape[2]

    x = x_ref[...].reshape(1, n)
    y = y_ref[...].reshape(1, n)
    z = z_ref[...].reshape(1, n)
    qx = qx_ref[...].reshape(qb, 1)
    qy = qy_ref[...].reshape(qb, 1)
    qz = qz_ref[...].reshape(qb, 1)

    dx = qx - x
    dy = qy - y
    dz = qz - z
    d2 = (dx * dx + dy * dy) + dz * dz
    m_s[...] = (d2 <= r2).astype(_F32)

    u = (lax.broadcasted_iota(jnp.int32, (_CH, _CH), 0)
         <= lax.broadcasted_iota(jnp.int32, (_CH, _CH), 1)).astype(_F32)

    def ch_body(c, base):
        mc = m_s[:, pl.ds(c * _CH, _CH)]
        loc = _dot(mc, u)
        r_s[:, pl.ds(c * _CH, _CH)] = loc + base
        return base + loc[:, _CH - 1:_CH]

    lax.fori_loop(0, n // _CH, ch_body, jnp.zeros((qb, 1), _F32))

    count = r_s[:, n - 1:n]
    r_s[...] = jnp.where(m_s[...] > 0.0, r_s[...], 0.0)

    fx_hi, fx_lo = _split(fx_ref[...].reshape(n, d1))
    qw = qw_ref[...].reshape(qb, d1)
    w2h, w2l = _split(w2_ref[...])
    b2 = b2_ref[...]
    w3h, w3l = _split(w3_ref[...])
    b3 = b3_ref[...]

    def k_body(k, acc):
        kf = (k + 1).astype(_F32)
        oh = (r_s[...] == kf).astype(_F32)
        g = _dot(oh, fx_hi) + _dot(oh, fx_lo)
        h1 = jnp.maximum(g - qw, 0.0)
        h2 = jnp.maximum(_mm3(h1, w2h, w2l) + b2, 0.0)
        h3 = jnp.maximum(_mm3(h2, w3h, w3l) + b3, 0.0)
        valid = count > k.astype(_F32)
        return jnp.maximum(acc, jnp.where(valid, h3, -jnp.inf))

    kmax = jnp.minimum(jnp.max(count), float(nsample)).astype(jnp.int32)
    acc0 = jnp.full((qb, d3), -jnp.inf, _F32)
    acc = lax.fori_loop(0, kmax, k_body, acc0)
    o_ref[...] = acc.reshape(1, qb, d3)


def _sa(xyz, new_xyz, fx, qw, w2, b2, w3, b3, radius, qblk):
    b, n, _ = xyz.shape
    q = new_xyz.shape[1]
    d1 = fx.shape[2]
    d2 = w2.shape[1]
    d3 = w3.shape[1]
    X = xyz[..., 0].reshape(b, 1, n)
    Y = xyz[..., 1].reshape(b, 1, n)
    Z = xyz[..., 2].reshape(b, 1, n)
    nblk = q // qblk
    QX = new_xyz[..., 0].reshape(b, nblk, qblk, 1)
    QY = new_xyz[..., 1].reshape(b, nblk, qblk, 1)
    QZ = new_xyz[..., 2].reshape(b, nblk, qblk, 1)
    grid = (b, nblk)
    return pl.pallas_call(
        functools.partial(_sa_body, radius * radius, _NSAMPLE),
        grid=grid,
        in_specs=[
            pl.BlockSpec((1, 1, n), lambda i, j: (i, 0, 0)),
            pl.BlockSpec((1, 1, n), lambda i, j: (i, 0, 0)),
            pl.BlockSpec((1, 1, n), lambda i, j: (i, 0, 0)),
            pl.BlockSpec((1, 1, qblk, 1), lambda i, j: (i, j, 0, 0)),
            pl.BlockSpec((1, 1, qblk, 1), lambda i, j: (i, j, 0, 0)),
            pl.BlockSpec((1, 1, qblk, 1), lambda i, j: (i, j, 0, 0)),
            pl.BlockSpec((1, n, d1), lambda i, j: (i, 0, 0)),
            pl.BlockSpec((1, qblk, d1), lambda i, j: (i, j, 0)),
            pl.BlockSpec((d1, d2), lambda i, j: (0, 0)),
            pl.BlockSpec((1, d2), lambda i, j: (0, 0)),
            pl.BlockSpec((d2, d3), lambda i, j: (0, 0)),
            pl.BlockSpec((1, d3), lambda i, j: (0, 0)),
        ],
        out_specs=pl.BlockSpec((1, qblk, d3), lambda i, j: (i, j, 0)),
        out_shape=jax.ShapeDtypeStruct((b, q, d3), _F32),
        scratch_shapes=[pltpu.VMEM((qblk, n), _F32),
                        pltpu.VMEM((qblk, n), _F32)],
    )(X, Y, Z, QX, QY, QZ, fx, qw, w2, b2.reshape(1, d2),
      w3, b3.reshape(1, d3))


def _fp_body(ux_ref, uy_ref, uz_ref, kx_ref, ky_ref, kz_ref,
             f_ref, w_ref, b_ref, o_ref):
    qb = ux_ref.shape[2]
    n = kx_ref.shape[2]
    c = f_ref.shape[2]

    ux = ux_ref[...].reshape(qb, 1)
    uy = uy_ref[...].reshape(qb, 1)
    uz = uz_ref[...].reshape(qb, 1)
    kx = kx_ref[...].reshape(1, n)
    ky = ky_ref[...].reshape(1, n)
    kz = kz_ref[...].reshape(1, n)

    dx = ux - kx
    dy = uy - ky
    dz = uz - kz
    d2 = (dx * dx + dy * dy) + dz * dz
    iota = lax.broadcasted_iota(jnp.int32, (qb, n), 1)

    m1 = jnp.min(d2, axis=1, keepdims=True)
    i1 = jnp.min(jnp.where(d2 == m1, iota, n), axis=1, keepdims=True)
    d2b = jnp.where(iota == i1, 1e30, d2)
    m2 = jnp.min(d2b, axis=1, keepdims=True)
    i2 = jnp.min(jnp.where(d2b == m2, iota, n), axis=1, keepdims=True)
    d2c = jnp.where(iota == i2, 1e30, d2b)
    m3 = jnp.min(d2c, axis=1, keepdims=True)
    i3 = jnp.min(jnp.where(d2c == m3, iota, n), axis=1, keepdims=True)

    w1 = 1.0 / (m1 + 1e-8)
    w2 = 1.0 / (m2 + 1e-8)
    w3 = 1.0 / (m3 + 1e-8)
    tot = (w1 + w2) + w3
    w1 = w1 / tot
    w2 = w2 / tot
    w3 = w3 / tot

    s = (jnp.where(iota == i1, w1, 0.0) + jnp.where(iota == i2, w2, 0.0)
         + jnp.where(iota == i3, w3, 0.0))
    fh, fl = _split(f_ref[...].reshape(n, c))
    interp = _mm3(s, fh, fl)
    wh, wl = _split(w_ref[...])
    up = jnp.maximum(_mm3(interp, wh, wl) + b_ref[...], 0.0)
    o_ref[...] = up.reshape(o_ref.shape)


def _fp(unknown, known, known_feats, w, bias, qblk):
    b, nq, _ = unknown.shape
    n = known.shape[1]
    c = known_feats.shape[2]
    d = w.shape[1]
    nblk = nq // qblk
    UX = unknown[..., 0].reshape(b, nblk, qblk, 1)
    UY = unknown[..., 1].reshape(b, nblk, qblk, 1)
    UZ = unknown[..., 2].reshape(b, nblk, qblk, 1)
    KX = known[..., 0].reshape(b, 1, n)
    KY = known[..., 1].reshape(b, 1, n)
    KZ = known[..., 2].reshape(b, 1, n)
    grid = (b, nblk)
    return pl.pallas_call(
        _fp_body,
        grid=grid,
        in_specs=[
            pl.BlockSpec((1, 1, qblk, 1), lambda i, j: (i, j, 0, 0)),
            pl.BlockSpec((1, 1, qblk, 1), lambda i, j: (i, j, 0, 0)),
            pl.BlockSpec((1, 1, qblk, 1), lambda i, j: (i, j, 0, 0)),
            pl.BlockSpec((1, 1, n), lambda i, j: (i, 0, 0)),
            pl.BlockSpec((1, 1, n), lambda i, j: (i, 0, 0)),
            pl.BlockSpec((1, 1, n), lambda i, j: (i, 0, 0)),
            pl.BlockSpec((1, n, c), lambda i, j: (i, 0, 0)),
            pl.BlockSpec((c, d), lambda i, j: (0, 0)),
            pl.BlockSpec((1, d), lambda i, j: (0, 0)),
        ],
        out_specs=pl.BlockSpec((1, qblk, d), lambda i, j: (i, j, 0)),
        out_shape=jax.ShapeDtypeStruct((b, nq, d), _F32),
    )(UX, UY, UZ, KX, KY, KZ, known_feats, w, bias.reshape(1, d))


def kernel(points, sa_params, fp_params):
    xyz0 = points[..., :3]
    l_xyz = [xyz0]
    l_feats = [None]
    for k in range(4):
        xyz = l_xyz[k]
        feats = l_feats[k]
        (w1, b1), (w2, b2), (w3, b3) = sa_params[k]
        new_xyz = _fps(xyz, _NPOINTS[k])
        if feats is None:
            cand = xyz
        else:
            cand = jnp.concatenate([xyz, feats], axis=-1)
        fx = _linear(cand, w1, b1)
        qw = _linear(new_xyz, w1[:3], jnp.zeros((w1.shape[1],), _F32))
        qblk = min(_NPOINTS[k], 256)
        nf = _sa(xyz, new_xyz, fx, qw, w2, b2, w3, b3, _RADII[k], qblk)
        l_xyz.append(new_xyz)
        l_feats.append(nf)
    ups = []
    for k in range(3):
        (w, b), = fp_params[k]
        up = _fp(xyz0, l_xyz[k + 2], l_feats[k + 2], w, b, 512)
        ups.append(up)
    return jnp.concatenate([xyz0, l_feats[1]] + ups, axis=-1)
